# Initial kernel scaffold; baseline (speedup 1.0000x reference)
import jax
import jax.numpy as jnp
from jax.experimental import pallas as pl


def kernel(x, idx, W_conv, bn_gamma, bn_beta, W1, W2):
    raise NotImplementedError("write your pallas kernel here")


# speedup vs baseline: 7.8300x; 7.8300x over previous
import functools

import jax
import jax.numpy as jnp
from jax import lax
from jax.experimental import pallas as pl
from jax.experimental.pallas import tpu as pltpu
from jax.experimental.pallas import tpu_sc as plsc

B, C, N, K = 8, 64, 2048, 20
OUT = 64
MID = 16
EPS = 1e-5
NEG = 0.2
PTS = B * N
M_EDGES = PTS * K

NC, NS, L = 2, 16, 16
NW = NC * NS
PPW = PTS // NW
CP = 64
NCH = PPW // CP
RPC = CP * K
GW = 128
NSUB = RPC // GW
CVECS = OUT // L


def _proj_body(x_ref, wa_ref, wd_ref, p_ref, q_ref):
    xb = x_ref[0]
    dn = (((0,), (1,)), ((), ()))
    p_ref[0] = lax.dot_general(xb, wa_ref[...], dn,
                               preferred_element_type=jnp.float32)
    q_ref[0] = lax.dot_general(xb, wd_ref[...], dn,
                               preferred_element_type=jnp.float32)


def _project(x, wa, wd):
    return pl.pallas_call(
        _proj_body,
        grid=(B,),
        in_specs=[
            pl.BlockSpec((1, C, N), lambda b: (b, 0, 0)),
            pl.BlockSpec((OUT, C), lambda b: (0, 0)),
            pl.BlockSpec((OUT, C), lambda b: (0, 0)),
        ],
        out_specs=[
            pl.BlockSpec((1, N, OUT), lambda b: (b, 0, 0)),
            pl.BlockSpec((1, N, OUT), lambda b: (b, 0, 0)),
        ],
        out_shape=[
            jax.ShapeDtypeStruct((B, N, OUT), jnp.float32),
            jax.ShapeDtypeStruct((B, N, OUT), jnp.float32),
        ],
    )(x, wa, wd)


def _sc_body(p_hbm, idx_hbm, mx_hbm, s_hbm, ss_hbm,
             idx_v, rows_v, mx_v, s_v, ss_v, sem):
    wid = lax.axis_index("s") * NC + lax.axis_index("c")

    def chunk(ci, carry):
        pt0 = wid * PPW + ci * CP
        pltpu.sync_copy(idx_hbm.at[wid * NCH + ci], idx_v)
        cps = [pltpu.async_copy(p_hbm.at[idx_v.at[j]], rows_v.at[j], sem)
               for j in range(NSUB)]
        for cp_ in cps:
            cp_.wait()

        def pt(p, carry2):
            r0 = p * K

            def kb(k, acc):
                r = r0 + k
                j = r // GW
                i = r % GW
                new = []
                for c in range(CVECS):
                    v = rows_v[j, i, pl.ds(c * L, L)]
                    new.append(jnp.maximum(acc[c], v))
                for c in range(CVECS):
                    v = rows_v[j, i, pl.ds(c * L, L)]
                    new.append(acc[CVECS + c] + v)
                for c in range(CVECS):
                    v = rows_v[j, i, pl.ds(c * L, L)]
                    new.append(acc[2 * CVECS + c] + v * v)
                return tuple(new)

            ninf = jnp.full((L,), -jnp.inf, jnp.float32)
            zero = jnp.zeros((L,), jnp.float32)
            acc = lax.fori_loop(0, K, kb,
                                (ninf,) * CVECS + (zero,) * (2 * CVECS))
            for c in range(CVECS):
                mx_v[p, pl.ds(c * L, L)] = acc[c]
                s_v[p, pl.ds(c * L, L)] = acc[CVECS + c]
                ss_v[p, pl.ds(c * L, L)] = acc[2 * CVECS + c]
            return carry2

        lax.fori_loop(0, CP, pt, 0)
        pltpu.sync_copy(mx_v, mx_hbm.at[pl.ds(pt0, CP)])
        pltpu.sync_copy(s_v, s_hbm.at[pl.ds(pt0, CP)])
        pltpu.sync_copy(ss_v, ss_hbm.at[pl.ds(pt0, CP)])
        return carry

    lax.fori_loop(0, NCH, chunk, 0)


@functools.lru_cache(maxsize=1)
def _sc_gather_reduce():
    mesh = plsc.VectorSubcoreMesh(core_axis_name="c", subcore_axis_name="s",
                                  num_cores=NC, num_subcores=NS)
    return pl.kernel(
        _sc_body,
        out_type=(
            jax.ShapeDtypeStruct((PTS, OUT), jnp.float32),
            jax.ShapeDtypeStruct((PTS, OUT), jnp.float32),
            jax.ShapeDtypeStruct((PTS, OUT), jnp.float32),
        ),
        mesh=mesh,
        scratch_types=[
            pltpu.VMEM((NSUB, GW), jnp.int32),
            pltpu.VMEM((NSUB, GW, OUT), jnp.float32),
            pltpu.VMEM((CP, OUT), jnp.float32),
            pltpu.VMEM((CP, OUT), jnp.float32),
            pltpu.VMEM((CP, OUT), jnp.float32),
            pltpu.SemaphoreType.DMA,
        ],
        compiler_params=pltpu.CompilerParams(use_tc_tiling_on_sc=False),
    )


def _fin_body(q_ref, mx_ref, s_ref, ss_ref, g_ref, b_ref, w1_ref, w2_ref,
              eye_ref, o_ref):
    q = q_ref[...]
    s = s_ref[...]
    sum_s = jnp.sum(s, axis=0, keepdims=True)
    sum_ss = jnp.sum(ss_ref[...], axis=0, keepdims=True)
    cross = jnp.sum(s * q, axis=0, keepdims=True)
    sum_q = jnp.sum(q, axis=0, keepdims=True)
    sum_qq = jnp.sum(q * q, axis=0, keepdims=True)
    inv = 1.0 / M_EDGES
    mean = (sum_s + K * sum_q) * inv
    e2 = (sum_ss + 2.0 * cross + K * sum_qq) * inv
    var = e2 - mean * mean
    scale = g_ref[...] * lax.rsqrt(var + EPS)
    shift = b_ref[...] - mean * scale
    act = (q + mx_ref[...]) * scale + shift
    act = jnp.where(act >= 0, act, NEG * act)

    dn = (((1,), (1,)), ((), ()))
    wm = jnp.concatenate(
        [jnp.mean(lax.slice(act, (b * N, 0), ((b + 1) * N, OUT)),
                  axis=0, keepdims=True) for b in range(B)], axis=0)
    h1 = jnp.maximum(
        lax.dot_general(wm, w1_ref[...], dn,
                        preferred_element_type=jnp.float32), 0.0)
    gate = jax.nn.sigmoid(
        lax.dot_general(h1, w2_ref[...], dn,
                        preferred_element_type=jnp.float32))
    eye = eye_ref[...]
    for b in range(B):
        ab = lax.slice(act, (b * N, 0), ((b + 1) * N, OUT))
        gb = lax.slice(gate, (b, 0), (b + 1, OUT))
        o_ref[b] = lax.dot_general(eye, ab * gb, dn,
                                   preferred_element_type=jnp.float32)


def _finalize(q2, mx2, s2, ss2, gamma, beta, w1, w2, eye):
    return pl.pallas_call(
        _fin_body,
        out_shape=jax.ShapeDtypeStruct((B, OUT, N), jnp.float32),
    )(q2, mx2, s2, ss2, gamma, beta, w1, w2, eye)


def kernel(x, idx, W_conv, bn_gamma, bn_beta, W1, W2):
    wa = W_conv[:, :C]
    wd = W_conv[:, C:] - wa
    p3, q3 = _project(x, wa, wd)
    p2 = p3.reshape(PTS, OUT)
    q2 = q3.reshape(PTS, OUT)
    offs = (jnp.arange(B, dtype=jnp.int32) * N).reshape(B, 1, 1)
    idx2 = (idx.astype(jnp.int32) + offs).reshape(NW * NCH, NSUB, GW)
    mx2, s2, ss2 = _sc_gather_reduce()(p2, idx2)
    return _finalize(q2, mx2, s2, ss2,
                     bn_gamma.reshape(1, OUT), bn_beta.reshape(1, OUT),
                     W1, W2, jnp.eye(OUT, dtype=jnp.float32))


# speedup vs baseline: 10.2555x; 1.3098x over previous
import functools

import jax
import jax.numpy as jnp
from jax import lax
from jax.experimental import pallas as pl
from jax.experimental.pallas import tpu as pltpu
from jax.experimental.pallas import tpu_sc as plsc

B, C, N, K = 8, 64, 2048, 20
OUT = 64
MID = 16
EPS = 1e-5
NEG = 0.2
PTS = B * N
M_EDGES = PTS * K

NC, NS, L = 2, 16, 16
NW = NC * NS
PPW = PTS // NW
CP = 32
NCH = PPW // CP
RPC = CP * K
GW = 128
NSUB = RPC // GW
CVECS = OUT // L


def _proj_body(x_ref, wa_ref, wd_ref, p_ref, q_ref):
    xb = x_ref[0]
    dn = (((0,), (1,)), ((), ()))
    p_ref[0] = lax.dot_general(xb, wa_ref[...], dn,
                               preferred_element_type=jnp.float32)
    q_ref[0] = lax.dot_general(xb, wd_ref[...], dn,
                               preferred_element_type=jnp.float32)


def _project(x, wa, wd):
    return pl.pallas_call(
        _proj_body,
        grid=(B,),
        in_specs=[
            pl.BlockSpec((1, C, N), lambda b: (b, 0, 0)),
            pl.BlockSpec((OUT, C), lambda b: (0, 0)),
            pl.BlockSpec((OUT, C), lambda b: (0, 0)),
        ],
        out_specs=[
            pl.BlockSpec((1, N, OUT), lambda b: (b, 0, 0)),
            pl.BlockSpec((1, N, OUT), lambda b: (b, 0, 0)),
        ],
        out_shape=[
            jax.ShapeDtypeStruct((B, N, OUT), jnp.float32),
            jax.ShapeDtypeStruct((B, N, OUT), jnp.float32),
        ],
    )(x, wa, wd)


def _sc_body(p_hbm, idx_hbm, mx_hbm, s_hbm, ss_hbm,
             idx_v, rows_v, mx_v, s_v, ss_v, sem0, sem1):
    wid = lax.axis_index("s") * NC + lax.axis_index("c")
    sems = (sem0, sem1)

    def fire(ci, buf):
        pltpu.sync_copy(idx_hbm.at[wid * NCH + ci], idx_v.at[buf])
        for j in range(NSUB):
            pltpu.async_copy(p_hbm.at[idx_v.at[buf, j]],
                             rows_v.at[buf, pl.ds(j * GW, GW)], sems[buf])

    def drain(buf):
        pltpu.make_async_copy(p_hbm.at[pl.ds(0, RPC)],
                              rows_v.at[buf], sems[buf]).wait()

    def compute(ci, buf):
        pt0 = wid * PPW + ci * CP

        def pt(p, carry):
            r0 = p * K
            mx = [None] * CVECS
            sa = [None] * CVECS
            qa = [None] * CVECS
            for c in range(CVECS):
                v = rows_v[buf, r0, pl.ds(c * L, L)]
                mx[c] = v
                sa[c] = v
                qa[c] = v * v
            for k in range(1, K):
                for c in range(CVECS):
                    v = rows_v[buf, r0 + k, pl.ds(c * L, L)]
                    mx[c] = jnp.maximum(mx[c], v)
                    sa[c] = sa[c] + v
                    qa[c] = qa[c] + v * v
            for c in range(CVECS):
                mx_v[p, pl.ds(c * L, L)] = mx[c]
                s_v[p, pl.ds(c * L, L)] = sa[c]
                ss_v[p, pl.ds(c * L, L)] = qa[c]
            return carry

        lax.fori_loop(0, CP, pt, 0)
        pltpu.sync_copy(mx_v, mx_hbm.at[pl.ds(pt0, CP)])
        pltpu.sync_copy(s_v, s_hbm.at[pl.ds(pt0, CP)])
        pltpu.sync_copy(ss_v, ss_hbm.at[pl.ds(pt0, CP)])

    fire(0, 0)

    def body2(m, carry):
        ci = m * 2
        drain(0)
        fire(ci + 1, 1)
        compute(ci, 0)
        drain(1)

        @pl.when(ci + 2 < NCH)
        def _fire_next():
            fire(ci + 2, 0)

        compute(ci + 1, 1)
        return carry

    lax.fori_loop(0, NCH // 2, body2, 0)


@functools.lru_cache(maxsize=1)
def _sc_gather_reduce():
    mesh = plsc.VectorSubcoreMesh(core_axis_name="c", subcore_axis_name="s",
                                  num_cores=NC, num_subcores=NS)
    return pl.kernel(
        _sc_body,
        out_type=(
            jax.ShapeDtypeStruct((PTS, OUT), jnp.float32),
            jax.ShapeDtypeStruct((PTS, OUT), jnp.float32),
            jax.ShapeDtypeStruct((PTS, OUT), jnp.float32),
        ),
        mesh=mesh,
        scratch_types=[
            pltpu.VMEM((2, NSUB, GW), jnp.int32),
            pltpu.VMEM((2, RPC, OUT), jnp.float32),
            pltpu.VMEM((CP, OUT), jnp.float32),
            pltpu.VMEM((CP, OUT), jnp.float32),
            pltpu.VMEM((CP, OUT), jnp.float32),
            pltpu.SemaphoreType.DMA,
            pltpu.SemaphoreType.DMA,
        ],
        compiler_params=pltpu.CompilerParams(use_tc_tiling_on_sc=False),
    )


def _fin_body(q_ref, mx_ref, s_ref, ss_ref, g_ref, b_ref, w1_ref, w2_ref,
              eye_ref, o_ref):
    q = q_ref[...]
    s = s_ref[...]
    sum_s = jnp.sum(s, axis=0, keepdims=True)
    sum_ss = jnp.sum(ss_ref[...], axis=0, keepdims=True)
    cross = jnp.sum(s * q, axis=0, keepdims=True)
    sum_q = jnp.sum(q, axis=0, keepdims=True)
    sum_qq = jnp.sum(q * q, axis=0, keepdims=True)
    inv = 1.0 / M_EDGES
    mean = (sum_s + K * sum_q) * inv
    e2 = (sum_ss + 2.0 * cross + K * sum_qq) * inv
    var = e2 - mean * mean
    scale = g_ref[...] * lax.rsqrt(var + EPS)
    shift = b_ref[...] - mean * scale
    act = (q + mx_ref[...]) * scale + shift
    act = jnp.where(act >= 0, act, NEG * act)

    dn = (((1,), (1,)), ((), ()))
    wm = jnp.concatenate(
        [jnp.mean(lax.slice(act, (b * N, 0), ((b + 1) * N, OUT)),
                  axis=0, keepdims=True) for b in range(B)], axis=0)
    h1 = jnp.maximum(
        lax.dot_general(wm, w1_ref[...], dn,
                        preferred_element_type=jnp.float32), 0.0)
    gate = jax.nn.sigmoid(
        lax.dot_general(h1, w2_ref[...], dn,
                        preferred_element_type=jnp.float32))
    eye = eye_ref[...]
    for b in range(B):
        ab = lax.slice(act, (b * N, 0), ((b + 1) * N, OUT))
        gb = lax.slice(gate, (b, 0), (b + 1, OUT))
        o_ref[b] = lax.dot_general(eye, ab * gb, dn,
                                   preferred_element_type=jnp.float32)


def _finalize(q2, mx2, s2, ss2, gamma, beta, w1, w2, eye):
    return pl.pallas_call(
        _fin_body,
        out_shape=jax.ShapeDtypeStruct((B, OUT, N), jnp.float32),
    )(q2, mx2, s2, ss2, gamma, beta, w1, w2, eye)


def kernel(x, idx, W_conv, bn_gamma, bn_beta, W1, W2):
    wa = W_conv[:, :C]
    wd = W_conv[:, C:] - wa
    p3, q3 = _project(x, wa, wd)
    p2 = p3.reshape(PTS, OUT)
    q2 = q3.reshape(PTS, OUT)
    offs = (jnp.arange(B, dtype=jnp.int32) * N).reshape(B, 1, 1)
    idx2 = (idx.astype(jnp.int32) + offs).reshape(NW * NCH, NSUB, GW)
    mx2, s2, ss2 = _sc_gather_reduce()(p2, idx2)
    return _finalize(q2, mx2, s2, ss2,
                     bn_gamma.reshape(1, OUT), bn_beta.reshape(1, OUT),
                     W1, W2, jnp.eye(OUT, dtype=jnp.float32))
